# Initial kernel scaffold; baseline (speedup 1.0000x reference)
import jax
import jax.numpy as jnp
from jax.experimental import pallas as pl


def kernel(x, edge_list, batch, W1, b1, Wmid, bmid, Wfc, bfc):
    raise NotImplementedError("write your pallas kernel here")


# speedup vs baseline: 14.7330x; 14.7330x over previous
import jax
import jax.numpy as jnp
from jax import lax
from jax.experimental import pallas as pl
from jax.experimental.pallas import tpu as pltpu
from jax.experimental.pallas import tpu_sc as plsc

_N = 10000
_E = 320000
_F = 128
_OUT = 16
_B = 64

_NC = 2
_NS = 16
_NW = _NC * _NS
_EPW = _E // _NW
_K = 80
_NCH = _EPW // _K
_NP = 10112
_STRIPE = _NP // _NS
_DRS = (80,) * 7 + (72,)
_CW = 128

_mesh = plsc.VectorSubcoreMesh(core_axis_name="c", subcore_axis_name="s")


def _sc_agg_body(src_hbm, dst_hbm, g_hbm, out_hbm,
                 sidx0, sidx1, didx0, didx1, rows0, rows1, accum, sem0, sem1):
    c = lax.axis_index("c")
    s = lax.axis_index("s")
    w = s * _NC + c
    base = w * _EPW
    r0 = s * _STRIPE

    def _zrow(i, carry):
        for j in range(_F // 16):
            rows0[i, pl.ds(j * 16, 16)] = jnp.zeros((16,), jnp.float32)
        return carry
    lax.fori_loop(0, _K, _zrow, 0)
    off = 0
    for sz in _DRS:
        pltpu.sync_copy(rows0.at[pl.ds(0, sz)], accum.at[pl.ds(r0 + off, sz)])
        off += sz

    sidxs = (sidx0, sidx1)
    didxs = (didx0, didx1)
    bufs = (rows0, rows1)
    sems = (sem0, sem1)

    plsc.subcore_barrier()

    def _issue(t, b):
        pltpu.sync_copy(src_hbm.at[pl.ds(base + t * _K, _K)], sidxs[b])
        pltpu.sync_copy(dst_hbm.at[pl.ds(base + t * _K, _K)], didxs[b])
        pltpu.async_copy(g_hbm.at[sidxs[b]], bufs[b], sems[b])

    def _finish(b):
        pltpu.make_async_copy(g_hbm.at[sidxs[b]], bufs[b], sems[b]).wait()
        pltpu.sync_copy(bufs[b], accum.at[didxs[b]], add=True)

    for b in range(2):
        _issue(b, b)

    def _ring(i2, carry):
        i = 2 * i2
        for b in range(2):
            _finish(b)
            _issue(i + b + 2, b)
        return carry
    lax.fori_loop(0, (_NCH - 3) // 2, _ring, 0)
    _finish(0)
    _issue(_NCH - 1, 0)
    _finish(1)
    _finish(0)

    plsc.subcore_barrier()
    off = 0
    for sz in _DRS:
        pltpu.sync_copy(accum.at[pl.ds(r0 + off, sz)], rows0.at[pl.ds(0, sz)])
        pltpu.sync_copy(rows0.at[pl.ds(0, sz)],
                        out_hbm.at[c, pl.ds(r0 + off, sz)])
        off += sz


_sc_agg = pl.kernel(
    _sc_agg_body,
    out_type=jax.ShapeDtypeStruct((_NC, _NP, _F), jnp.float32),
    mesh=_mesh,
    scratch_types=[
        pltpu.VMEM((_K,), jnp.int32),
        pltpu.VMEM((_K,), jnp.int32),
        pltpu.VMEM((_K,), jnp.int32),
        pltpu.VMEM((_K,), jnp.int32),
        pltpu.VMEM((_K, _F), jnp.float32),
        pltpu.VMEM((_K, _F), jnp.float32),
        pltpu.VMEM_SHARED((_NP, _F), jnp.float32),
        pltpu.SemaphoreType.DMA,
        pltpu.SemaphoreType.DMA,
    ],
)


def _sc_cnt_body(dst_hbm, out_hbm, didx, ones_v, dbuf, accum):
    c = lax.axis_index("c")
    s = lax.axis_index("s")
    w = s * _NC + c
    base = w * _EPW

    def _zrow(i, carry):
        for j in range(_CW // 16):
            dbuf[i, pl.ds(j * 16, 16)] = jnp.zeros((16,), jnp.float32)
            ones_v[i, pl.ds(j * 16, 16)] = jnp.ones((16,), jnp.float32)
        return carry
    lax.fori_loop(0, _K, _zrow, 0)

    for k in range(_NS):
        @pl.when(s == k)
        def _zc():
            off = k * _STRIPE
            for sz in _DRS:
                pltpu.sync_copy(dbuf.at[pl.ds(0, sz)],
                                accum.at[pl.ds(off, sz)])
                off += sz

    plsc.subcore_barrier()

    def _chunk(t, carry):
        pltpu.sync_copy(dst_hbm.at[pl.ds(base + t * _K, _K)], didx)
        pltpu.sync_copy(ones_v, accum.at[didx], add=True)
        return carry
    lax.fori_loop(0, _NCH, _chunk, 0)

    plsc.subcore_barrier()
    for k in range(_NS):
        @pl.when(s == k)
        def _dr():
            off = k * _STRIPE
            for sz in _DRS:
                pltpu.sync_copy(accum.at[pl.ds(off, sz)],
                                dbuf.at[pl.ds(0, sz)])
                pltpu.sync_copy(dbuf.at[pl.ds(0, sz)],
                                out_hbm.at[c, pl.ds(off, sz)])
                off += sz


_sc_cnt = pl.kernel(
    _sc_cnt_body,
    out_type=jax.ShapeDtypeStruct((_NC, _NP, _CW), jnp.float32),
    mesh=_mesh,
    scratch_types=[
        pltpu.VMEM((_K,), jnp.int32),
        pltpu.VMEM((_K, _CW), jnp.float32),
        pltpu.VMEM((_K, _CW), jnp.float32),
        pltpu.VMEM_SHARED((_NP, _CW), jnp.float32),
    ],
)


_R = 1000
_G = _N // _R


def _tc_pre_body(cnt_ref, x_ref, w1_ref, dis_ref, gs_ref):
    cnt = cnt_ref[0, :, 0:1] + cnt_ref[1, :, 0:1]
    dis = lax.rsqrt(cnt + 1.0)
    g = jnp.dot(x_ref[...], w1_ref[...], preferred_element_type=jnp.float32)
    dis_ref[...] = dis
    gs_ref[...] = g * dis


def _tc_pre(cnt, x, W1):
    return pl.pallas_call(
        _tc_pre_body,
        grid=(_G,),
        in_specs=[
            pl.BlockSpec((_NC, _R, _CW), lambda i: (0, i, 0)),
            pl.BlockSpec((_R, _F), lambda i: (i, 0)),
            pl.BlockSpec((_F, _F), lambda i: (0, 0)),
        ],
        out_specs=[
            pl.BlockSpec((_R, 1), lambda i: (i, 0)),
            pl.BlockSpec((_R, _F), lambda i: (i, 0)),
        ],
        out_shape=[
            jax.ShapeDtypeStruct((_N, 1), jnp.float32),
            jax.ShapeDtypeStruct((_N, _F), jnp.float32),
        ],
    )(cnt, x, W1)


def _tc_mid_body(a_ref, gs_ref, dis_ref, b_ref, w_ref, out_ref):
    dis = dis_ref[...]
    t = (a_ref[0] + a_ref[1] + gs_ref[...]) * dis + b_ref[...]
    h = jnp.maximum(t, 0.0)
    out_ref[...] = jnp.dot(h, w_ref[...], preferred_element_type=jnp.float32) * dis


def _tc_mid(a, gs, dis, b, W):
    return pl.pallas_call(
        _tc_mid_body,
        grid=(_G,),
        in_specs=[
            pl.BlockSpec((_NC, _R, _F), lambda i: (0, i, 0)),
            pl.BlockSpec((_R, _F), lambda i: (i, 0)),
            pl.BlockSpec((_R, 1), lambda i: (i, 0)),
            pl.BlockSpec((1, _F), lambda i: (0, 0)),
            pl.BlockSpec((_F, _F), lambda i: (0, 0)),
        ],
        out_specs=pl.BlockSpec((_R, _F), lambda i: (i, 0)),
        out_shape=jax.ShapeDtypeStruct((_N, _F), jnp.float32),
    )(a, gs, dis, b, W)


def _tc_fin_body(a_ref, gs_ref, dis_ref, b_ref, batch_ref, wfc_ref, bfc_ref,
                 out_ref, acc, cacc):
    i = pl.program_id(0)

    @pl.when(i == 0)
    def _init():
        acc[...] = jnp.zeros_like(acc)
        cacc[...] = jnp.zeros_like(cacc)

    t = (a_ref[0] + a_ref[1] + gs_ref[...]) * dis_ref[...] + b_ref[...]
    h = jnp.maximum(t, 0.0)
    seg = batch_ref[...]
    ids = lax.broadcasted_iota(jnp.int32, (_R, _B), 1)
    m = (seg == ids).astype(jnp.float32)
    acc[...] += lax.dot_general(m, h, (((0,), (0,)), ((), ())),
                                preferred_element_type=jnp.float32)
    ones = jnp.ones((_R, 1), jnp.float32)
    cacc[...] += lax.dot_general(m, ones, (((0,), (0,)), ((), ())),
                                 preferred_element_type=jnp.float32)

    @pl.when(i == pl.num_programs(0) - 1)
    def _fin():
        pooled = acc[...] / jnp.maximum(cacc[...], 1.0)
        logits = jnp.dot(pooled, wfc_ref[...],
                         preferred_element_type=jnp.float32) + bfc_ref[...]
        mx = jnp.max(logits, axis=1, keepdims=True)
        z = logits - mx
        lse = jnp.log(jnp.sum(jnp.exp(z), axis=1, keepdims=True))
        out_ref[...] = z - lse


def _tc_fin(a, gs, dis, b, batch2, Wfc, bfc):
    return pl.pallas_call(
        _tc_fin_body,
        grid=(_G,),
        in_specs=[
            pl.BlockSpec((_NC, _R, _F), lambda i: (0, i, 0)),
            pl.BlockSpec((_R, _F), lambda i: (i, 0)),
            pl.BlockSpec((_R, 1), lambda i: (i, 0)),
            pl.BlockSpec((1, _F), lambda i: (0, 0)),
            pl.BlockSpec((_R, 1), lambda i: (i, 0)),
            pl.BlockSpec((_F, _OUT), lambda i: (0, 0)),
            pl.BlockSpec((1, _OUT), lambda i: (0, 0)),
        ],
        out_specs=pl.BlockSpec((_B, _OUT), lambda i: (0, 0)),
        out_shape=jax.ShapeDtypeStruct((_B, _OUT), jnp.float32),
        scratch_shapes=[
            pltpu.VMEM((_B, _F), jnp.float32),
            pltpu.VMEM((_B, 1), jnp.float32),
        ],
    )(a, gs, dis, b, batch2, Wfc, bfc)


def kernel(x, edge_list, batch, W1, b1, Wmid, bmid, Wfc, bfc):
    src = edge_list[0].astype(jnp.int32)
    dst = edge_list[1].astype(jnp.int32)
    batch2 = batch.astype(jnp.int32).reshape(_N, 1)

    cnt = _sc_cnt(dst)
    dis, gs = _tc_pre(cnt, x, W1)

    biases = [b1.reshape(1, _F)] + [bmid[i].reshape(1, _F) for i in range(4)]
    weights = [Wmid[i] for i in range(4)]

    for l in range(4):
        a = _sc_agg(src, dst, gs)
        gs = _tc_mid(a, gs, dis, biases[l], weights[l])
    a = _sc_agg(src, dst, gs)
    return _tc_fin(a, gs, dis, biases[4], batch2, Wfc, bfc.reshape(1, _OUT))


# speedup vs baseline: 21.8006x; 1.4797x over previous
import jax
import jax.numpy as jnp
from jax import lax
from jax.experimental import pallas as pl
from jax.experimental.pallas import tpu as pltpu
from jax.experimental.pallas import tpu_sc as plsc

_N = 10000
_E = 320000
_F = 128
_OUT = 16
_B = 64

_NC = 2
_NS = 16
_NW = _NC * _NS
_EPW = _E // _NW
_K = 80
_NCH = _EPW // _K
_NP = 10112
_STRIPE = _NP // _NS
_DRS = (80,) * 7 + (72,)
_CW = 128

_mesh = plsc.VectorSubcoreMesh(core_axis_name="c", subcore_axis_name="s")


def _sc_agg_body(src_hbm, dst_hbm, g_hbm, out_hbm,
                 sidx, didx, rows0, rows1, accum, sem0, sem1):
    c = lax.axis_index("c")
    s = lax.axis_index("s")
    w = s * _NC + c
    r0 = s * _STRIPE

    def _zrow(i, carry):
        for j in range(_F // 16):
            rows0[i, pl.ds(j * 16, 16)] = jnp.zeros((16,), jnp.float32)
        return carry
    lax.fori_loop(0, _K, _zrow, 0)
    off = 0
    for sz in _DRS:
        pltpu.sync_copy(rows0.at[pl.ds(0, sz)], accum.at[pl.ds(r0 + off, sz)])
        off += sz

    pltpu.sync_copy(src_hbm.at[pl.ds(w * _EPW, _EPW)], sidx)
    pltpu.sync_copy(dst_hbm.at[pl.ds(w * _EPW, _EPW)], didx)

    bufs = (rows0, rows1)
    sems = (sem0, sem1)

    plsc.subcore_barrier()

    def _issue(t, b):
        pltpu.async_copy(g_hbm.at[sidx.at[pl.ds(t * _K, _K)]], bufs[b],
                         sems[b])

    def _finish(t, b):
        pltpu.make_async_copy(g_hbm.at[sidx.at[pl.ds(t * _K, _K)]], bufs[b],
                              sems[b]).wait()
        pltpu.sync_copy(bufs[b], accum.at[didx.at[pl.ds(t * _K, _K)]],
                        add=True)

    for b in range(2):
        _issue(b, b)

    def _ring(i2, carry):
        i = 2 * i2
        for b in range(2):
            _finish(i + b, b)
            _issue(i + b + 2, b)
        return carry
    lax.fori_loop(0, (_NCH - 3) // 2, _ring, 0)
    _finish(_NCH - 3, 0)
    _issue(_NCH - 1, 0)
    _finish(_NCH - 2, 1)
    _finish(_NCH - 1, 0)

    plsc.subcore_barrier()
    off = 0
    for sz in _DRS:
        pltpu.sync_copy(accum.at[pl.ds(r0 + off, sz)], rows0.at[pl.ds(0, sz)])
        pltpu.sync_copy(rows0.at[pl.ds(0, sz)],
                        out_hbm.at[c, pl.ds(r0 + off, sz)])
        off += sz


_sc_agg = pl.kernel(
    _sc_agg_body,
    out_type=jax.ShapeDtypeStruct((_NC, _NP, _F), jnp.float32),
    mesh=_mesh,
    scratch_types=[
        pltpu.VMEM((_EPW,), jnp.int32),
        pltpu.VMEM((_EPW,), jnp.int32),
        pltpu.VMEM((_K, _F), jnp.float32),
        pltpu.VMEM((_K, _F), jnp.float32),
        pltpu.VMEM_SHARED((_NP, _F), jnp.float32),
        pltpu.SemaphoreType.DMA,
        pltpu.SemaphoreType.DMA,
    ],
)


def _sc_cnt_body(dst_hbm, out_hbm, didx, ones_v, dbuf, accum):
    c = lax.axis_index("c")
    s = lax.axis_index("s")
    w = s * _NC + c

    def _zrow(i, carry):
        for j in range(_CW // 16):
            dbuf[i, pl.ds(j * 16, 16)] = jnp.zeros((16,), jnp.float32)
            ones_v[i, pl.ds(j * 16, 16)] = jnp.ones((16,), jnp.float32)
        return carry
    lax.fori_loop(0, _K, _zrow, 0)
    pltpu.sync_copy(dst_hbm.at[pl.ds(w * _EPW, _EPW)], didx)

    for k in range(_NS):
        @pl.when(s == k)
        def _zc():
            off = k * _STRIPE
            for sz in _DRS:
                pltpu.sync_copy(dbuf.at[pl.ds(0, sz)],
                                accum.at[pl.ds(off, sz)])
                off += sz

    plsc.subcore_barrier()

    def _chunk(t, carry):
        pltpu.sync_copy(ones_v, accum.at[didx.at[pl.ds(t * _K, _K)]],
                        add=True)
        return carry
    lax.fori_loop(0, _NCH, _chunk, 0)

    plsc.subcore_barrier()
    for k in range(_NS):
        @pl.when(s == k)
        def _dr():
            off = k * _STRIPE
            for sz in _DRS:
                pltpu.sync_copy(accum.at[pl.ds(off, sz)],
                                dbuf.at[pl.ds(0, sz)])
                pltpu.sync_copy(dbuf.at[pl.ds(0, sz)],
                                out_hbm.at[c, pl.ds(off, sz)])
                off += sz


_sc_cnt = pl.kernel(
    _sc_cnt_body,
    out_type=jax.ShapeDtypeStruct((_NC, _NP, _CW), jnp.float32),
    mesh=_mesh,
    scratch_types=[
        pltpu.VMEM((_EPW,), jnp.int32),
        pltpu.VMEM((_K, _CW), jnp.float32),
        pltpu.VMEM((_K, _CW), jnp.float32),
        pltpu.VMEM_SHARED((_NP, _CW), jnp.float32),
    ],
)


_R = 1000
_G = _N // _R


def _tc_pre_body(cnt_ref, x_ref, w1_ref, dis_ref, gs_ref):
    cnt = cnt_ref[0, :, 0:1] + cnt_ref[1, :, 0:1]
    dis = lax.rsqrt(cnt + 1.0)
    g = jnp.dot(x_ref[...], w1_ref[...], preferred_element_type=jnp.float32)
    dis_ref[...] = dis
    gs_ref[...] = g * dis


def _tc_pre(cnt, x, W1):
    return pl.pallas_call(
        _tc_pre_body,
        grid=(_G,),
        in_specs=[
            pl.BlockSpec((_NC, _R, _CW), lambda i: (0, i, 0)),
            pl.BlockSpec((_R, _F), lambda i: (i, 0)),
            pl.BlockSpec((_F, _F), lambda i: (0, 0)),
        ],
        out_specs=[
            pl.BlockSpec((_R, 1), lambda i: (i, 0)),
            pl.BlockSpec((_R, _F), lambda i: (i, 0)),
        ],
        out_shape=[
            jax.ShapeDtypeStruct((_N, 1), jnp.float32),
            jax.ShapeDtypeStruct((_N, _F), jnp.float32),
        ],
    )(cnt, x, W1)


def _tc_mid_body(a_ref, gs_ref, dis_ref, b_ref, w_ref, out_ref):
    dis = dis_ref[...]
    t = (a_ref[0] + a_ref[1] + gs_ref[...]) * dis + b_ref[...]
    h = jnp.maximum(t, 0.0)
    out_ref[...] = jnp.dot(h, w_ref[...], preferred_element_type=jnp.float32) * dis


def _tc_mid(a, gs, dis, b, W):
    return pl.pallas_call(
        _tc_mid_body,
        grid=(_G,),
        in_specs=[
            pl.BlockSpec((_NC, _R, _F), lambda i: (0, i, 0)),
            pl.BlockSpec((_R, _F), lambda i: (i, 0)),
            pl.BlockSpec((_R, 1), lambda i: (i, 0)),
            pl.BlockSpec((1, _F), lambda i: (0, 0)),
            pl.BlockSpec((_F, _F), lambda i: (0, 0)),
        ],
        out_specs=pl.BlockSpec((_R, _F), lambda i: (i, 0)),
        out_shape=jax.ShapeDtypeStruct((_N, _F), jnp.float32),
    )(a, gs, dis, b, W)


def _tc_fin_body(a_ref, gs_ref, dis_ref, b_ref, batch_ref, wfc_ref, bfc_ref,
                 out_ref, acc, cacc):
    i = pl.program_id(0)

    @pl.when(i == 0)
    def _init():
        acc[...] = jnp.zeros_like(acc)
        cacc[...] = jnp.zeros_like(cacc)

    t = (a_ref[0] + a_ref[1] + gs_ref[...]) * dis_ref[...] + b_ref[...]
    h = jnp.maximum(t, 0.0)
    seg = batch_ref[...]
    ids = lax.broadcasted_iota(jnp.int32, (_R, _B), 1)
    m = (seg == ids).astype(jnp.float32)
    acc[...] += lax.dot_general(m, h, (((0,), (0,)), ((), ())),
                                preferred_element_type=jnp.float32)
    ones = jnp.ones((_R, 1), jnp.float32)
    cacc[...] += lax.dot_general(m, ones, (((0,), (0,)), ((), ())),
                                 preferred_element_type=jnp.float32)

    @pl.when(i == pl.num_programs(0) - 1)
    def _fin():
        pooled = acc[...] / jnp.maximum(cacc[...], 1.0)
        logits = jnp.dot(pooled, wfc_ref[...],
                         preferred_element_type=jnp.float32) + bfc_ref[...]
        mx = jnp.max(logits, axis=1, keepdims=True)
        z = logits - mx
        lse = jnp.log(jnp.sum(jnp.exp(z), axis=1, keepdims=True))
        out_ref[...] = z - lse


def _tc_fin(a, gs, dis, b, batch2, Wfc, bfc):
    return pl.pallas_call(
        _tc_fin_body,
        grid=(_G,),
        in_specs=[
            pl.BlockSpec((_NC, _R, _F), lambda i: (0, i, 0)),
            pl.BlockSpec((_R, _F), lambda i: (i, 0)),
            pl.BlockSpec((_R, 1), lambda i: (i, 0)),
            pl.BlockSpec((1, _F), lambda i: (0, 0)),
            pl.BlockSpec((_R, 1), lambda i: (i, 0)),
            pl.BlockSpec((_F, _OUT), lambda i: (0, 0)),
            pl.BlockSpec((1, _OUT), lambda i: (0, 0)),
        ],
        out_specs=pl.BlockSpec((_B, _OUT), lambda i: (0, 0)),
        out_shape=jax.ShapeDtypeStruct((_B, _OUT), jnp.float32),
        scratch_shapes=[
            pltpu.VMEM((_B, _F), jnp.float32),
            pltpu.VMEM((_B, 1), jnp.float32),
        ],
    )(a, gs, dis, b, batch2, Wfc, bfc)


def kernel(x, edge_list, batch, W1, b1, Wmid, bmid, Wfc, bfc):
    src = edge_list[0].astype(jnp.int32)
    dst = edge_list[1].astype(jnp.int32)
    batch2 = batch.astype(jnp.int32).reshape(_N, 1)

    cnt = _sc_cnt(dst)
    dis, gs = _tc_pre(cnt, x, W1)

    biases = [b1.reshape(1, _F)] + [bmid[i].reshape(1, _F) for i in range(4)]
    weights = [Wmid[i] for i in range(4)]

    for l in range(4):
        a = _sc_agg(src, dst, gs)
        gs = _tc_mid(a, gs, dis, biases[l], weights[l])
    a = _sc_agg(src, dst, gs)
    return _tc_fin(a, gs, dis, biases[4], batch2, Wfc, bfc.reshape(1, _OUT))


# speedup vs baseline: 23.0572x; 1.0576x over previous
import jax
import jax.numpy as jnp
from jax import lax
from jax.experimental import pallas as pl
from jax.experimental.pallas import tpu as pltpu
from jax.experimental.pallas import tpu_sc as plsc

_N = 10000
_E = 320000
_F = 128
_OUT = 16
_B = 64

_NC = 2
_NS = 16
_NW = _NC * _NS
_EPW = _E // _NW
_K = 80
_NCH = _EPW // _K
_NP = 10112
_STRIPE = _NP // _NS
_DRS = (80,) * 7 + (72,)
_CW = 16

_mesh = plsc.VectorSubcoreMesh(core_axis_name="c", subcore_axis_name="s")


def _sc_agg_body(src_hbm, dst_hbm, g_hbm, out_hbm,
                 sidx, didx, rows0, rows1, accum, sem0, sem1):
    c = lax.axis_index("c")
    s = lax.axis_index("s")
    w = s * _NC + c
    r0 = s * _STRIPE

    def _zrow(i, carry):
        for j in range(_F // 16):
            rows0[i, pl.ds(j * 16, 16)] = jnp.zeros((16,), jnp.float32)
        return carry
    lax.fori_loop(0, _K, _zrow, 0)
    off = 0
    for sz in _DRS:
        pltpu.sync_copy(rows0.at[pl.ds(0, sz)], accum.at[pl.ds(r0 + off, sz)])
        off += sz

    pltpu.sync_copy(src_hbm.at[pl.ds(w * _EPW, _EPW)], sidx)
    pltpu.sync_copy(dst_hbm.at[pl.ds(w * _EPW, _EPW)], didx)

    bufs = (rows0, rows1)
    sems = (sem0, sem1)

    plsc.subcore_barrier()

    def _issue(t, b):
        pltpu.async_copy(g_hbm.at[sidx.at[pl.ds(t * _K, _K)]], bufs[b],
                         sems[b])

    def _finish(t, b):
        pltpu.make_async_copy(g_hbm.at[sidx.at[pl.ds(t * _K, _K)]], bufs[b],
                              sems[b]).wait()
        pltpu.sync_copy(bufs[b], accum.at[didx.at[pl.ds(t * _K, _K)]],
                        add=True)

    for b in range(2):
        _issue(b, b)

    def _ring(i2, carry):
        i = 2 * i2
        for b in range(2):
            _finish(i + b, b)
            _issue(i + b + 2, b)
        return carry
    lax.fori_loop(0, (_NCH - 3) // 2, _ring, 0)
    _finish(_NCH - 3, 0)
    _issue(_NCH - 1, 0)
    _finish(_NCH - 2, 1)
    _finish(_NCH - 1, 0)

    plsc.subcore_barrier()
    off = 0
    for sz in _DRS:
        pltpu.sync_copy(accum.at[pl.ds(r0 + off, sz)], rows0.at[pl.ds(0, sz)])
        pltpu.sync_copy(rows0.at[pl.ds(0, sz)],
                        out_hbm.at[c, pl.ds(r0 + off, sz)])
        off += sz


_sc_agg = pl.kernel(
    _sc_agg_body,
    out_type=jax.ShapeDtypeStruct((_NC, _NP, _F), jnp.float32),
    mesh=_mesh,
    scratch_types=[
        pltpu.VMEM((_EPW,), jnp.int32),
        pltpu.VMEM((_EPW,), jnp.int32),
        pltpu.VMEM((_K, _F), jnp.float32),
        pltpu.VMEM((_K, _F), jnp.float32),
        pltpu.VMEM_SHARED((_NP, _F), jnp.float32),
        pltpu.SemaphoreType.DMA,
        pltpu.SemaphoreType.DMA,
    ],
)


def _sc_cnt_body(dst_hbm, out_hbm, didx, ones_v, dbuf, accum):
    c = lax.axis_index("c")
    s = lax.axis_index("s")
    w = s * _NC + c

    def _zrow(i, carry):
        dbuf[i, pl.ds(0, 16)] = jnp.zeros((16,), jnp.float32)
        ones_v[i, pl.ds(0, 16)] = jnp.ones((16,), jnp.float32)
        return carry
    lax.fori_loop(0, _K, _zrow, 0)
    pltpu.sync_copy(dst_hbm.at[pl.ds(w * _EPW, _EPW)], didx)

    for k in range(_NS):
        @pl.when(s == k)
        def _zc():
            off = k * _STRIPE
            for sz in _DRS:
                pltpu.sync_copy(dbuf.at[pl.ds(0, sz)],
                                accum.at[pl.ds(off, sz)])
                off += sz

    plsc.subcore_barrier()

    def _chunk(t, carry):
        pltpu.sync_copy(ones_v, accum.at[didx.at[pl.ds(t * _K, _K)]],
                        add=True)
        return carry
    lax.fori_loop(0, _NCH, _chunk, 0)

    plsc.subcore_barrier()
    for k in range(_NS):
        @pl.when(s == k)
        def _dr():
            off = k * _STRIPE
            for sz in _DRS:
                pltpu.sync_copy(accum.at[pl.ds(off, sz)],
                                dbuf.at[pl.ds(0, sz)])
                pltpu.sync_copy(dbuf.at[pl.ds(0, sz)],
                                out_hbm.at[c, pl.ds(off, sz)])
                off += sz


_sc_cnt = pl.kernel(
    _sc_cnt_body,
    out_type=jax.ShapeDtypeStruct((_NC, _NP, _CW), jnp.float32),
    mesh=_mesh,
    scratch_types=[
        pltpu.VMEM((_EPW,), jnp.int32),
        pltpu.VMEM((_K, _CW), jnp.float32),
        pltpu.VMEM((_K, _CW), jnp.float32),
        pltpu.VMEM_SHARED((_NP, _CW), jnp.float32),
    ],
)


_R = 1000
_G = _N // _R


def _tc_mm_body(x_ref, w1_ref, g_ref):
    g_ref[...] = jnp.dot(x_ref[...], w1_ref[...],
                         preferred_element_type=jnp.float32)


def _tc_mm(x, W1):
    return pl.pallas_call(
        _tc_mm_body,
        grid=(_G,),
        in_specs=[
            pl.BlockSpec((_R, _F), lambda i: (i, 0)),
            pl.BlockSpec((_F, _F), lambda i: (0, 0)),
        ],
        out_specs=pl.BlockSpec((_R, _F), lambda i: (i, 0)),
        out_shape=jax.ShapeDtypeStruct((_N, _F), jnp.float32),
    )(x, W1)


def _tc_pre_body(cnt_ref, g_ref, dis_ref, gs_ref):
    cnt = cnt_ref[0, :, 0:1] + cnt_ref[1, :, 0:1]
    dis = lax.rsqrt(cnt + 1.0)
    dis_ref[...] = dis
    gs_ref[...] = g_ref[...] * dis


def _tc_pre(cnt, g):
    return pl.pallas_call(
        _tc_pre_body,
        grid=(_G,),
        in_specs=[
            pl.BlockSpec((_NC, _R, _CW), lambda i: (0, i, 0)),
            pl.BlockSpec((_R, _F), lambda i: (i, 0)),
        ],
        out_specs=[
            pl.BlockSpec((_R, 1), lambda i: (i, 0)),
            pl.BlockSpec((_R, _F), lambda i: (i, 0)),
        ],
        out_shape=[
            jax.ShapeDtypeStruct((_N, 1), jnp.float32),
            jax.ShapeDtypeStruct((_N, _F), jnp.float32),
        ],
    )(cnt, g)


def _tc_mid_body(a_ref, gs_ref, dis_ref, b_ref, w_ref, out_ref):
    dis = dis_ref[...]
    t = (a_ref[0] + a_ref[1] + gs_ref[...]) * dis + b_ref[...]
    h = jnp.maximum(t, 0.0)
    out_ref[...] = jnp.dot(h, w_ref[...], preferred_element_type=jnp.float32) * dis


def _tc_mid(a, gs, dis, b, W):
    return pl.pallas_call(
        _tc_mid_body,
        grid=(_G,),
        in_specs=[
            pl.BlockSpec((_NC, _R, _F), lambda i: (0, i, 0)),
            pl.BlockSpec((_R, _F), lambda i: (i, 0)),
            pl.BlockSpec((_R, 1), lambda i: (i, 0)),
            pl.BlockSpec((1, _F), lambda i: (0, 0)),
            pl.BlockSpec((_F, _F), lambda i: (0, 0)),
        ],
        out_specs=pl.BlockSpec((_R, _F), lambda i: (i, 0)),
        out_shape=jax.ShapeDtypeStruct((_N, _F), jnp.float32),
    )(a, gs, dis, b, W)


def _tc_fin_body(a_ref, gs_ref, dis_ref, b_ref, batch_ref, wfc_ref, bfc_ref,
                 out_ref, acc, cacc):
    i = pl.program_id(0)

    @pl.when(i == 0)
    def _init():
        acc[...] = jnp.zeros_like(acc)
        cacc[...] = jnp.zeros_like(cacc)

    t = (a_ref[0] + a_ref[1] + gs_ref[...]) * dis_ref[...] + b_ref[...]
    h = jnp.maximum(t, 0.0)
    seg = batch_ref[...]
    ids = lax.broadcasted_iota(jnp.int32, (_R, _B), 1)
    m = (seg == ids).astype(jnp.float32)
    acc[...] += lax.dot_general(m, h, (((0,), (0,)), ((), ())),
                                preferred_element_type=jnp.float32)
    ones = jnp.ones((_R, 1), jnp.float32)
    cacc[...] += lax.dot_general(m, ones, (((0,), (0,)), ((), ())),
                                 preferred_element_type=jnp.float32)

    @pl.when(i == pl.num_programs(0) - 1)
    def _fin():
        pooled = acc[...] / jnp.maximum(cacc[...], 1.0)
        logits = jnp.dot(pooled, wfc_ref[...],
                         preferred_element_type=jnp.float32) + bfc_ref[...]
        mx = jnp.max(logits, axis=1, keepdims=True)
        z = logits - mx
        lse = jnp.log(jnp.sum(jnp.exp(z), axis=1, keepdims=True))
        out_ref[...] = z - lse


def _tc_fin(a, gs, dis, b, batch2, Wfc, bfc):
    return pl.pallas_call(
        _tc_fin_body,
        grid=(_G,),
        in_specs=[
            pl.BlockSpec((_NC, _R, _F), lambda i: (0, i, 0)),
            pl.BlockSpec((_R, _F), lambda i: (i, 0)),
            pl.BlockSpec((_R, 1), lambda i: (i, 0)),
            pl.BlockSpec((1, _F), lambda i: (0, 0)),
            pl.BlockSpec((_R, 1), lambda i: (i, 0)),
            pl.BlockSpec((_F, _OUT), lambda i: (0, 0)),
            pl.BlockSpec((1, _OUT), lambda i: (0, 0)),
        ],
        out_specs=pl.BlockSpec((_B, _OUT), lambda i: (0, 0)),
        out_shape=jax.ShapeDtypeStruct((_B, _OUT), jnp.float32),
        scratch_shapes=[
            pltpu.VMEM((_B, _F), jnp.float32),
            pltpu.VMEM((_B, 1), jnp.float32),
        ],
    )(a, gs, dis, b, batch2, Wfc, bfc)


def kernel(x, edge_list, batch, W1, b1, Wmid, bmid, Wfc, bfc):
    src = edge_list[0].astype(jnp.int32)
    dst = edge_list[1].astype(jnp.int32)
    batch2 = batch.astype(jnp.int32).reshape(_N, 1)

    cnt = _sc_cnt(dst)
    g = _tc_mm(x, W1)
    dis, gs = _tc_pre(cnt, g)

    biases = [b1.reshape(1, _F)] + [bmid[i].reshape(1, _F) for i in range(4)]
    weights = [Wmid[i] for i in range(4)]

    for l in range(4):
        a = _sc_agg(src, dst, gs)
        gs = _tc_mid(a, gs, dis, biases[l], weights[l])
    a = _sc_agg(src, dst, gs)
    return _tc_fin(a, gs, dis, biases[4], batch2, Wfc, bfc.reshape(1, _OUT))


# speedup vs baseline: 23.3060x; 1.0108x over previous
import jax
import jax.numpy as jnp
from jax import lax
from jax.experimental import pallas as pl
from jax.experimental.pallas import tpu as pltpu
from jax.experimental.pallas import tpu_sc as plsc

_N = 10000
_E = 320000
_F = 128
_OUT = 16
_B = 64

_NC = 2
_NS = 16
_NW = _NC * _NS
_EPW = _E // _NW
_K = 80
_NCH = _EPW // _K
_NP = 10112
_STRIPE = _NP // _NS
_DRS = (80,) * 7 + (72,)
_CW = 16

_mesh = plsc.VectorSubcoreMesh(core_axis_name="c", subcore_axis_name="s")


def _sc_agg_body(src_hbm, dst_hbm, g_hbm, out_hbm,
                 sidx, didx, rows0, rows1, accum, sem0, sem1):
    c = lax.axis_index("c")
    s = lax.axis_index("s")
    w = s * _NC + c
    r0 = s * _STRIPE

    def _zrow(i, carry):
        for j in range(_F // 16):
            rows0[i, pl.ds(j * 16, 16)] = jnp.zeros((16,), jnp.float32)
        return carry
    lax.fori_loop(0, _K, _zrow, 0)
    off = 0
    for sz in _DRS:
        pltpu.sync_copy(rows0.at[pl.ds(0, sz)], accum.at[pl.ds(r0 + off, sz)])
        off += sz

    pltpu.sync_copy(src_hbm.at[pl.ds(w * _EPW, _EPW)], sidx)
    pltpu.sync_copy(dst_hbm.at[pl.ds(w * _EPW, _EPW)], didx)

    bufs = (rows0, rows1)
    sems = (sem0, sem1)

    plsc.subcore_barrier()

    def _issue(t, b):
        pltpu.async_copy(g_hbm.at[sidx.at[pl.ds(t * _K, _K)]], bufs[b],
                         sems[b])

    def _finish(t, b):
        pltpu.make_async_copy(g_hbm.at[sidx.at[pl.ds(t * _K, _K)]], bufs[b],
                              sems[b]).wait()
        pltpu.sync_copy(bufs[b], accum.at[didx.at[pl.ds(t * _K, _K)]],
                        add=True)

    for b in range(2):
        _issue(b, b)

    def _ring(i2, carry):
        i = 2 * i2
        for b in range(2):
            _finish(i + b, b)
            _issue(i + b + 2, b)
        return carry
    lax.fori_loop(0, (_NCH - 3) // 2, _ring, 0)
    _finish(_NCH - 3, 0)
    _issue(_NCH - 1, 0)
    _finish(_NCH - 2, 1)
    _finish(_NCH - 1, 0)

    plsc.subcore_barrier()
    pltpu.sync_copy(accum.at[pl.ds(r0, _STRIPE)],
                    out_hbm.at[c, pl.ds(r0, _STRIPE)])


_sc_agg = pl.kernel(
    _sc_agg_body,
    out_type=jax.ShapeDtypeStruct((_NC, _NP, _F), jnp.float32),
    mesh=_mesh,
    scratch_types=[
        pltpu.VMEM((_EPW,), jnp.int32),
        pltpu.VMEM((_EPW,), jnp.int32),
        pltpu.VMEM((_K, _F), jnp.float32),
        pltpu.VMEM((_K, _F), jnp.float32),
        pltpu.VMEM_SHARED((_NP, _F), jnp.float32),
        pltpu.SemaphoreType.DMA,
        pltpu.SemaphoreType.DMA,
    ],
)


def _sc_cnt_body(dst_hbm, out_hbm, didx, ones_v, dbuf, accum):
    c = lax.axis_index("c")
    s = lax.axis_index("s")
    w = s * _NC + c

    def _zrow(i, carry):
        dbuf[i, pl.ds(0, 16)] = jnp.zeros((16,), jnp.float32)
        ones_v[i, pl.ds(0, 16)] = jnp.ones((16,), jnp.float32)
        return carry
    lax.fori_loop(0, _K, _zrow, 0)
    pltpu.sync_copy(dst_hbm.at[pl.ds(w * _EPW, _EPW)], didx)

    r0 = s * _STRIPE
    off = 0
    for sz in _DRS:
        pltpu.sync_copy(dbuf.at[pl.ds(0, sz)],
                        accum.at[pl.ds(r0 + off, sz)])
        off += sz

    plsc.subcore_barrier()

    def _chunk(t, carry):
        pltpu.sync_copy(ones_v, accum.at[didx.at[pl.ds(t * _K, _K)]],
                        add=True)
        return carry
    lax.fori_loop(0, _NCH, _chunk, 0)

    plsc.subcore_barrier()
    pltpu.sync_copy(accum.at[pl.ds(r0, _STRIPE)],
                    out_hbm.at[c, pl.ds(r0, _STRIPE)])


_sc_cnt = pl.kernel(
    _sc_cnt_body,
    out_type=jax.ShapeDtypeStruct((_NC, _NP, _CW), jnp.float32),
    mesh=_mesh,
    scratch_types=[
        pltpu.VMEM((_EPW,), jnp.int32),
        pltpu.VMEM((_K, _CW), jnp.float32),
        pltpu.VMEM((_K, _CW), jnp.float32),
        pltpu.VMEM_SHARED((_NP, _CW), jnp.float32),
    ],
)


_R = 1000
_G = _N // _R


def _tc_mm_body(x_ref, w1_ref, g_ref):
    g_ref[...] = jnp.dot(x_ref[...], w1_ref[...],
                         preferred_element_type=jnp.float32)


def _tc_mm(x, W1):
    return pl.pallas_call(
        _tc_mm_body,
        grid=(_G,),
        in_specs=[
            pl.BlockSpec((_R, _F), lambda i: (i, 0)),
            pl.BlockSpec((_F, _F), lambda i: (0, 0)),
        ],
        out_specs=pl.BlockSpec((_R, _F), lambda i: (i, 0)),
        out_shape=jax.ShapeDtypeStruct((_N, _F), jnp.float32),
    )(x, W1)


def _tc_pre_body(cnt_ref, g_ref, dis_ref, gs_ref):
    cnt = cnt_ref[0, :, 0:1] + cnt_ref[1, :, 0:1]
    dis = lax.rsqrt(cnt + 1.0)
    dis_ref[...] = dis
    gs_ref[...] = g_ref[...] * dis


def _tc_pre(cnt, g):
    return pl.pallas_call(
        _tc_pre_body,
        grid=(_G,),
        in_specs=[
            pl.BlockSpec((_NC, _R, _CW), lambda i: (0, i, 0)),
            pl.BlockSpec((_R, _F), lambda i: (i, 0)),
        ],
        out_specs=[
            pl.BlockSpec((_R, 1), lambda i: (i, 0)),
            pl.BlockSpec((_R, _F), lambda i: (i, 0)),
        ],
        out_shape=[
            jax.ShapeDtypeStruct((_N, 1), jnp.float32),
            jax.ShapeDtypeStruct((_N, _F), jnp.float32),
        ],
    )(cnt, g)


def _tc_mid_body(a_ref, gs_ref, dis_ref, b_ref, w_ref, out_ref):
    dis = dis_ref[...]
    t = (a_ref[0] + a_ref[1] + gs_ref[...]) * dis + b_ref[...]
    h = jnp.maximum(t, 0.0)
    out_ref[...] = jnp.dot(h, w_ref[...], preferred_element_type=jnp.float32) * dis


def _tc_mid(a, gs, dis, b, W):
    return pl.pallas_call(
        _tc_mid_body,
        grid=(_G,),
        in_specs=[
            pl.BlockSpec((_NC, _R, _F), lambda i: (0, i, 0)),
            pl.BlockSpec((_R, _F), lambda i: (i, 0)),
            pl.BlockSpec((_R, 1), lambda i: (i, 0)),
            pl.BlockSpec((1, _F), lambda i: (0, 0)),
            pl.BlockSpec((_F, _F), lambda i: (0, 0)),
        ],
        out_specs=pl.BlockSpec((_R, _F), lambda i: (i, 0)),
        out_shape=jax.ShapeDtypeStruct((_N, _F), jnp.float32),
    )(a, gs, dis, b, W)


def _tc_fin_body(a_ref, gs_ref, dis_ref, b_ref, batch_ref, wfc_ref, bfc_ref,
                 out_ref, acc, cacc):
    i = pl.program_id(0)

    @pl.when(i == 0)
    def _init():
        acc[...] = jnp.zeros_like(acc)
        cacc[...] = jnp.zeros_like(cacc)

    t = (a_ref[0] + a_ref[1] + gs_ref[...]) * dis_ref[...] + b_ref[...]
    h = jnp.maximum(t, 0.0)
    seg = batch_ref[...]
    ids = lax.broadcasted_iota(jnp.int32, (_R, _B), 1)
    m = (seg == ids).astype(jnp.float32)
    acc[...] += lax.dot_general(m, h, (((0,), (0,)), ((), ())),
                                preferred_element_type=jnp.float32)
    ones = jnp.ones((_R, 1), jnp.float32)
    cacc[...] += lax.dot_general(m, ones, (((0,), (0,)), ((), ())),
                                 preferred_element_type=jnp.float32)

    @pl.when(i == pl.num_programs(0) - 1)
    def _fin():
        pooled = acc[...] / jnp.maximum(cacc[...], 1.0)
        logits = jnp.dot(pooled, wfc_ref[...],
                         preferred_element_type=jnp.float32) + bfc_ref[...]
        mx = jnp.max(logits, axis=1, keepdims=True)
        z = logits - mx
        lse = jnp.log(jnp.sum(jnp.exp(z), axis=1, keepdims=True))
        out_ref[...] = z - lse


def _tc_fin(a, gs, dis, b, batch2, Wfc, bfc):
    return pl.pallas_call(
        _tc_fin_body,
        grid=(_G,),
        in_specs=[
            pl.BlockSpec((_NC, _R, _F), lambda i: (0, i, 0)),
            pl.BlockSpec((_R, _F), lambda i: (i, 0)),
            pl.BlockSpec((_R, 1), lambda i: (i, 0)),
            pl.BlockSpec((1, _F), lambda i: (0, 0)),
            pl.BlockSpec((_R, 1), lambda i: (i, 0)),
            pl.BlockSpec((_F, _OUT), lambda i: (0, 0)),
            pl.BlockSpec((1, _OUT), lambda i: (0, 0)),
        ],
        out_specs=pl.BlockSpec((_B, _OUT), lambda i: (0, 0)),
        out_shape=jax.ShapeDtypeStruct((_B, _OUT), jnp.float32),
        scratch_shapes=[
            pltpu.VMEM((_B, _F), jnp.float32),
            pltpu.VMEM((_B, 1), jnp.float32),
        ],
    )(a, gs, dis, b, batch2, Wfc, bfc)


def kernel(x, edge_list, batch, W1, b1, Wmid, bmid, Wfc, bfc):
    src = edge_list[0].astype(jnp.int32)
    dst = edge_list[1].astype(jnp.int32)
    batch2 = batch.astype(jnp.int32).reshape(_N, 1)

    cnt = _sc_cnt(dst)
    g = _tc_mm(x, W1)
    dis, gs = _tc_pre(cnt, g)

    biases = [b1.reshape(1, _F)] + [bmid[i].reshape(1, _F) for i in range(4)]
    weights = [Wmid[i] for i in range(4)]

    for l in range(4):
        a = _sc_agg(src, dst, gs)
        gs = _tc_mid(a, gs, dis, biases[l], weights[l])
    a = _sc_agg(src, dst, gs)
    return _tc_fin(a, gs, dis, biases[4], batch2, Wfc, bfc.reshape(1, _OUT))
